# Initial kernel scaffold; baseline (speedup 1.0000x reference)
import jax
import jax.numpy as jnp
from jax.experimental import pallas as pl


def kernel(emb, W, b, batch_idx, dataset_idx):
    raise NotImplementedError("write your pallas kernel here")


# speedup vs baseline: 2.6478x; 2.6478x over previous
import jax
import jax.numpy as jnp
from jax.experimental import pallas as pl
from jax.experimental.pallas import tpu as pltpu

N = 8192
D_MODEL = 1024
OUT_DIM = 256
E = 8
G = 64
BN = 512
NB = N // BN


def _moe_block_kernel(bidx_ref, didx_ref, emb_ref, W_ref, b_ref, out_ref):
    bidx = bidx_ref[0]
    g_iota = jax.lax.broadcasted_iota(jnp.int32, (BN, G), 1)
    onehot = bidx == g_iota
    didx = didx_ref[...]
    x = emb_ref[...]
    for e in range(E):
        expert_graphs = didx == e
        mask = jnp.any(jnp.logical_and(onehot, expert_graphs),
                       axis=1, keepdims=True)
        out_ref[e] = jnp.zeros((BN, OUT_DIM), jnp.float32)

        @pl.when(jnp.any(mask))
        def _(e=e, mask=mask):
            y = jnp.dot(x, W_ref[e], preferred_element_type=jnp.float32)
            y = y + b_ref[pl.ds(e, 1), :]
            out_ref[e] = jnp.where(mask, y, 0.0)


def kernel(emb, W, b, batch_idx, dataset_idx):
    bidx = batch_idx.astype(jnp.int32).reshape(NB, BN, 1)
    didx = dataset_idx.astype(jnp.int32).reshape(1, G)
    out = pl.pallas_call(
        _moe_block_kernel,
        grid=(NB,),
        in_specs=[
            pl.BlockSpec((1, BN, 1), lambda i: (i, 0, 0)),
            pl.BlockSpec((1, G), lambda i: (0, 0)),
            pl.BlockSpec((BN, D_MODEL), lambda i: (i, 0)),
            pl.BlockSpec((E, D_MODEL, OUT_DIM), lambda i: (0, 0, 0)),
            pl.BlockSpec((E, OUT_DIM), lambda i: (0, 0)),
        ],
        out_specs=pl.BlockSpec((E, BN, OUT_DIM), lambda i: (0, i, 0)),
        out_shape=jax.ShapeDtypeStruct((E, N, OUT_DIM), jnp.float32),
        compiler_params=pltpu.CompilerParams(
            dimension_semantics=("arbitrary",),
        ),
    )(bidx, didx, emb, W, b)
    return out
